# Initial kernel scaffold; baseline (speedup 1.0000x reference)
import jax
import jax.numpy as jnp
from jax.experimental import pallas as pl


def kernel(idx, x, y, adj, W1, a_src1, a_dst1, b1, W2, a_src2, a_dst2, b2, fc1_w, fc1_b, fc2_w, fc2_b):
    raise NotImplementedError("write your pallas kernel here")


# speedup vs baseline: 11177.6109x; 11177.6109x over previous
import jax
import jax.numpy as jnp
from jax.experimental import pallas as pl

B, N, F_IN = 16, 512, 3
M = 64
HEADS, HID, OUT = 2, 3, 3
HSZ, NACT = 128, 512
YPAD = 256


def _gat_kernel(idx_ref, xT_ref, y_ref, adj_ref,
                W1T_ref, as1_ref, ad1_ref, b1_ref,
                W2T_ref, as2_ref, ad2_ref, b2_ref,
                f1i_ref, f1x_ref, f1y_ref, f1b_ref,
                f2w_ref, f2b_ref, out_ref):
    adjb = adj_ref[0]
    mask = adjb != 0.0
    xT = xT_ref[0]
    idx_row = idx_ref[0]
    y_row = y_ref[0]

    def att(hT, a_s_row, a_d_row):
        as_row = jnp.dot(a_s_row, hT, preferred_element_type=jnp.float32)
        ad_row = jnp.dot(a_d_row, hT, preferred_element_type=jnp.float32)
        as_col = as_row.reshape(N, 1)
        e = as_col + ad_row
        e = jnp.where(e > 0.0, e, 0.2 * e)
        e = jnp.where(mask, e, -1e30)
        m = jnp.max(e, axis=0, keepdims=True)
        ex = jnp.where(mask, jnp.exp(e - m), 0.0)
        den = jnp.sum(ex, axis=0, keepdims=True)
        alpha = ex / (den + 1e-16)
        return jnp.dot(hT, alpha, preferred_element_type=jnp.float32)

    h1T = jnp.dot(W1T_ref[...], xT, preferred_element_type=jnp.float32)
    o0 = att(h1T[0:HID], as1_ref[0:1], ad1_ref[0:1])
    o1 = att(h1T[HID:2 * HID], as1_ref[1:2], ad1_ref[1:2])
    h1o = jnp.concatenate([o0, o1], axis=0) + b1_ref[...]
    h1o = jnp.maximum(h1o, 0.0)

    h2T = jnp.dot(W2T_ref[...], h1o, preferred_element_type=jnp.float32)
    o2 = att(h2T, as2_ref[...], ad2_ref[...]) + b2_ref[...]

    p = jnp.dot(idx_row, f1i_ref[...], preferred_element_type=jnp.float32)
    p = p + jnp.dot(o2[0:1], f1x_ref[0], preferred_element_type=jnp.float32)
    p = p + jnp.dot(o2[1:2], f1x_ref[1], preferred_element_type=jnp.float32)
    p = p + jnp.dot(o2[2:3], f1x_ref[2], preferred_element_type=jnp.float32)
    p = p + jnp.dot(y_row, f1y_ref[...], preferred_element_type=jnp.float32)
    h = jnp.maximum(p + f1b_ref[...], 0.0)
    out_ref[0] = jnp.dot(h, f2w_ref[...],
                         preferred_element_type=jnp.float32) + f2b_ref[...]


def kernel(idx, x, y, adj, W1, a_src1, a_dst1, b1, W2, a_src2, a_dst2, b2,
           fc1_w, fc1_b, fc2_w, fc2_b):
    f32 = jnp.float32
    xT = x.transpose(0, 2, 1)
    idx3 = idx.reshape(B, 1, N)
    y_flat = y.reshape(B, 1, (M + 2) * 3)
    y_pad = jnp.zeros((B, 1, YPAD), f32).at[:, :, : (M + 2) * 3].set(y_flat)

    W1T = W1.T
    b1_col = b1.reshape(HEADS * HID, 1)
    W2T = W2.T
    b2_col = b2.reshape(OUT, 1)

    f1_idx = fc1_w[:, :N].T
    mid = fc1_w[:, N:N + OUT * N]
    f1_x = jnp.stack([mid[:, k::OUT].T for k in range(OUT)], 0)
    f1_y = jnp.zeros((YPAD, HSZ), f32).at[: (M + 2) * 3, :].set(
        fc1_w[:, N + OUT * N:].T)
    f1_b = fc1_b.reshape(1, HSZ)
    f2_w = fc2_w.T
    f2_b = fc2_b.reshape(1, NACT)

    def c0(ndim):
        return lambda b: (0,) * ndim

    in_specs = [
        pl.BlockSpec((1, 1, N), lambda b: (b, 0, 0)),
        pl.BlockSpec((1, F_IN, N), lambda b: (b, 0, 0)),
        pl.BlockSpec((1, 1, YPAD), lambda b: (b, 0, 0)),
        pl.BlockSpec((1, N, N), lambda b: (b, 0, 0)),
        pl.BlockSpec(W1T.shape, c0(2)),
        pl.BlockSpec(a_src1.shape, c0(2)),
        pl.BlockSpec(a_dst1.shape, c0(2)),
        pl.BlockSpec(b1_col.shape, c0(2)),
        pl.BlockSpec(W2T.shape, c0(2)),
        pl.BlockSpec(a_src2.shape, c0(2)),
        pl.BlockSpec(a_dst2.shape, c0(2)),
        pl.BlockSpec(b2_col.shape, c0(2)),
        pl.BlockSpec(f1_idx.shape, c0(2)),
        pl.BlockSpec(f1_x.shape, c0(3)),
        pl.BlockSpec(f1_y.shape, c0(2)),
        pl.BlockSpec(f1_b.shape, c0(2)),
        pl.BlockSpec(f2_w.shape, c0(2)),
        pl.BlockSpec(f2_b.shape, c0(2)),
    ]

    out = pl.pallas_call(
        _gat_kernel,
        grid=(B,),
        in_specs=in_specs,
        out_specs=pl.BlockSpec((1, 1, NACT), lambda b: (b, 0, 0)),
        out_shape=jax.ShapeDtypeStruct((B, 1, NACT), f32),
    )(idx3, xT, y_pad, adj, W1T, a_src1, a_dst1, b1_col,
      W2T, a_src2, a_dst2, b2_col, f1_idx, f1_x, f1_y, f1_b, f2_w, f2_b)
    return out.reshape(B, NACT)


# speedup vs baseline: 12812.3428x; 1.1463x over previous
import jax
import jax.numpy as jnp
from jax.experimental import pallas as pl
from jax.experimental.pallas import tpu as pltpu

B, N, F_IN = 16, 512, 3
M = 64
HEADS, HID, OUT = 2, 3, 3
HSZ, NACT = 128, 512
YPAD = 256


def _gat_kernel(idx_ref, xT_ref, y_ref, adj_ref,
                W1T_ref, as1_ref, ad1_ref, b1_ref,
                W2T_ref, as2_ref, ad2_ref, b2_ref,
                f1i_ref, f1x_ref, f1y_ref, f1b_ref,
                f2w_ref, f2b_ref, out_ref):
    adjb = adj_ref[0]
    maskf = jnp.where(adjb != 0.0, 0.0, -1e30)
    xT = xT_ref[0]
    idx_row = idx_ref[0]
    y_row = y_ref[0]

    def att(hT, a_s_row, a_d_row):
        as_row = jnp.dot(a_s_row, hT, preferred_element_type=jnp.float32)
        ad_row = jnp.dot(a_d_row, hT, preferred_element_type=jnp.float32)
        as_col = as_row.reshape(N, 1)
        e = as_col + ad_row
        e = jnp.where(e > 0.0, e, 0.2 * e) + maskf
        m = jnp.max(e, axis=0, keepdims=True)
        m = jnp.maximum(m, -60.0)
        ex = jnp.exp(e - m)
        den = jnp.sum(ex, axis=0, keepdims=True)
        o = jnp.dot(hT, ex, preferred_element_type=jnp.float32)
        return o / (den + 1e-16)

    h1T = jnp.dot(W1T_ref[...], xT, preferred_element_type=jnp.float32)
    o0 = att(h1T[0:HID], as1_ref[0:1], ad1_ref[0:1])
    o1 = att(h1T[HID:2 * HID], as1_ref[1:2], ad1_ref[1:2])
    h1o = jnp.concatenate([o0, o1], axis=0) + b1_ref[...]
    h1o = jnp.maximum(h1o, 0.0)

    h2T = jnp.dot(W2T_ref[...], h1o, preferred_element_type=jnp.float32)
    o2 = att(h2T, as2_ref[...], ad2_ref[...]) + b2_ref[...]

    p = jnp.dot(idx_row, f1i_ref[...], preferred_element_type=jnp.float32)
    p = p + jnp.dot(o2[0:1], f1x_ref[0], preferred_element_type=jnp.float32)
    p = p + jnp.dot(o2[1:2], f1x_ref[1], preferred_element_type=jnp.float32)
    p = p + jnp.dot(o2[2:3], f1x_ref[2], preferred_element_type=jnp.float32)
    p = p + jnp.dot(y_row, f1y_ref[...], preferred_element_type=jnp.float32)
    h = jnp.maximum(p + f1b_ref[...], 0.0)
    out_ref[0] = jnp.dot(h, f2w_ref[...],
                         preferred_element_type=jnp.float32) + f2b_ref[...]


def kernel(idx, x, y, adj, W1, a_src1, a_dst1, b1, W2, a_src2, a_dst2, b2,
           fc1_w, fc1_b, fc2_w, fc2_b):
    f32 = jnp.float32
    xT = x.transpose(0, 2, 1)
    idx3 = idx.reshape(B, 1, N)
    y_flat = y.reshape(B, 1, (M + 2) * 3)
    y_pad = jnp.zeros((B, 1, YPAD), f32).at[:, :, : (M + 2) * 3].set(y_flat)

    W1T = W1.T
    b1_col = b1.reshape(HEADS * HID, 1)
    W2T = W2.T
    b2_col = b2.reshape(OUT, 1)

    f1_idx = fc1_w[:, :N].T
    mid = fc1_w[:, N:N + OUT * N]
    f1_x = jnp.stack([mid[:, k::OUT].T for k in range(OUT)], 0)
    f1_y = jnp.zeros((YPAD, HSZ), f32).at[: (M + 2) * 3, :].set(
        fc1_w[:, N + OUT * N:].T)
    f1_b = fc1_b.reshape(1, HSZ)
    f2_w = fc2_w.T
    f2_b = fc2_b.reshape(1, NACT)

    def c0(ndim):
        return lambda b: (0,) * ndim

    in_specs = [
        pl.BlockSpec((1, 1, N), lambda b: (b, 0, 0)),
        pl.BlockSpec((1, F_IN, N), lambda b: (b, 0, 0)),
        pl.BlockSpec((1, 1, YPAD), lambda b: (b, 0, 0)),
        pl.BlockSpec((1, N, N), lambda b: (b, 0, 0)),
        pl.BlockSpec(W1T.shape, c0(2)),
        pl.BlockSpec(a_src1.shape, c0(2)),
        pl.BlockSpec(a_dst1.shape, c0(2)),
        pl.BlockSpec(b1_col.shape, c0(2)),
        pl.BlockSpec(W2T.shape, c0(2)),
        pl.BlockSpec(a_src2.shape, c0(2)),
        pl.BlockSpec(a_dst2.shape, c0(2)),
        pl.BlockSpec(b2_col.shape, c0(2)),
        pl.BlockSpec(f1_idx.shape, c0(2)),
        pl.BlockSpec(f1_x.shape, c0(3)),
        pl.BlockSpec(f1_y.shape, c0(2)),
        pl.BlockSpec(f1_b.shape, c0(2)),
        pl.BlockSpec(f2_w.shape, c0(2)),
        pl.BlockSpec(f2_b.shape, c0(2)),
    ]

    out = pl.pallas_call(
        _gat_kernel,
        grid=(B,),
        in_specs=in_specs,
        out_specs=pl.BlockSpec((1, 1, NACT), lambda b: (b, 0, 0)),
        out_shape=jax.ShapeDtypeStruct((B, 1, NACT), f32),
        compiler_params=pltpu.CompilerParams(
            dimension_semantics=("parallel",)),
    )(idx3, xT, y_pad, adj, W1T, a_src1, a_dst1, b1_col,
      W2T, a_src2, a_dst2, b2_col, f1_idx, f1_x, f1_y, f1_b, f2_w, f2_b)
    return out.reshape(B, NACT)
